# baseline (speedup 1.0000x reference)
import functools

import jax
import jax.numpy as jnp
from jax import lax
from jax.experimental import pallas as pl
from jax.experimental.pallas import tpu as pltpu
from jax.experimental.pallas import tpu_sc as plsc

_N_CELLS = 1024
_N_GENES_OI = 4096
_N_IN = 16
_N_OUT = 16
_D = _N_IN * _N_OUT


def _make_sc_gather(n_rows, d, num_workers, rows_per_worker, num_cores):
    def body(table_hbm, idx_hbm, out_hbm, idx_v, rows_v, sem):
        wid = lax.axis_index("s") * num_cores + lax.axis_index("c")
        base = wid * rows_per_worker
        pltpu.sync_copy(idx_hbm.at[pl.ds(base, rows_per_worker)], idx_v)
        pltpu.async_copy(table_hbm.at[idx_v], rows_v, sem).wait()
        pltpu.sync_copy(rows_v, out_hbm.at[pl.ds(base, rows_per_worker)])

    return pl.kernel(
        body,
        out_type=jax.ShapeDtypeStruct((n_rows, d), jnp.float32),
        mesh=plsc.VectorSubcoreMesh(core_axis_name="c", subcore_axis_name="s"),
        scratch_types=[
            pltpu.VMEM((rows_per_worker,), jnp.int32),
            pltpu.VMEM((rows_per_worker, d), jnp.float32),
            pltpu.SemaphoreType.DMA,
        ],
    )


def _matmul_body(x_ref, w_ref, b_ref, o_ref):
    k = pl.program_id(0)
    x3 = x_ref[...]
    xb = x3.reshape(x3.shape[0], x3.shape[1] * x3.shape[2]).astype(jnp.bfloat16)
    wb = w_ref[...].astype(jnp.bfloat16)
    acc = lax.dot_general(
        xb, wb, (((1,), (0,)), ((), ())), preferred_element_type=jnp.float32
    )

    @pl.when(k == 0)
    def _():
        o_ref[...] = acc + b_ref[...]

    @pl.when(k > 0)
    def _():
        o_ref[...] += acc


def kernel(cellgene_embedding, genes_oi, weight1, bias1):
    n_cells, n_genes_oi, n_in = cellgene_embedding.shape
    n_out = weight1.shape[2]
    d = n_in * n_out

    info = plsc.get_sparse_core_info()
    num_workers = info.num_cores * info.num_subcores
    rows_per_worker = n_genes_oi // num_workers

    table2d = weight1.reshape(weight1.shape[0], d)
    gather = _make_sc_gather(n_genes_oi, d, num_workers, rows_per_worker,
                             info.num_cores)
    w_rows = gather(table2d, genes_oi.astype(jnp.int32))

    big_k = n_genes_oi * n_in
    w2 = w_rows.reshape(big_k, n_out)
    bias2 = bias1.reshape(1, n_out)

    n_kk = big_k // 128
    x4 = cellgene_embedding.reshape(n_cells, n_kk, 128)

    bkk = 16
    bk = bkk * 128
    grid = (n_kk // bkk,)
    out = pl.pallas_call(
        _matmul_body,
        grid=grid,
        in_specs=[
            pl.BlockSpec((n_cells, bkk, 128), lambda k: (0, k, 0)),
            pl.BlockSpec((bk, n_out), lambda k: (k, 0)),
            pl.BlockSpec((1, n_out), lambda k: (0, 0)),
        ],
        out_specs=pl.BlockSpec((n_cells, n_out), lambda k: (0, 0)),
        out_shape=jax.ShapeDtypeStruct((n_cells, n_out), jnp.float32),
        compiler_params=pltpu.CompilerParams(
            dimension_semantics=("arbitrary",),
        ),
    )(x4, w2, bias2)
    return out


# speedup vs baseline: 3.1721x; 3.1721x over previous
import functools

import jax
import jax.numpy as jnp
from jax import lax
from jax.experimental import pallas as pl
from jax.experimental.pallas import tpu as pltpu
from jax.experimental.pallas import tpu_sc as plsc

_N_CELLS = 1024
_N_GENES_OI = 4096
_N_IN = 16
_N_OUT = 16
_D = _N_IN * _N_OUT


def _make_sc_gather(n_rows, d, num_workers, rows_per_worker, num_cores):
    def body(table_hbm, idx_hbm, out_hbm, idx_v, rows_v, sem):
        wid = lax.axis_index("s") * num_cores + lax.axis_index("c")
        base = wid * rows_per_worker
        pltpu.sync_copy(idx_hbm.at[pl.ds(base, rows_per_worker)], idx_v)
        pltpu.async_copy(table_hbm.at[idx_v], rows_v, sem).wait()
        pltpu.sync_copy(rows_v, out_hbm.at[pl.ds(base, rows_per_worker)])

    return pl.kernel(
        body,
        out_type=jax.ShapeDtypeStruct((n_rows, d), jnp.float32),
        mesh=plsc.VectorSubcoreMesh(core_axis_name="c", subcore_axis_name="s"),
        scratch_types=[
            pltpu.VMEM((rows_per_worker,), jnp.int32),
            pltpu.VMEM((rows_per_worker, d), jnp.float32),
            pltpu.SemaphoreType.DMA,
        ],
    )


def _matmul_body(n_in, n_out, x_ref, w_ref, b_ref, o_ref):
    bm = x_ref.shape[0]
    nc = n_in * n_out
    xb = x_ref[...].astype(jnp.bfloat16)
    wb = w_ref[...].astype(jnp.bfloat16)
    p = lax.dot_general(
        xb, wb, (((1,), (0,)), ((), ())), preferred_element_type=jnp.float32
    )
    rows = lax.broadcasted_iota(jnp.int32, p.shape, 0)
    lanes = lax.broadcasted_iota(jnp.int32, p.shape, 1)
    z = jnp.where((rows % n_in) == (lanes // n_out), p, 0.0).astype(jnp.bfloat16)
    f = (
        lax.broadcasted_iota(jnp.int32, (nc, n_out), 0) % n_out
        == lax.broadcasted_iota(jnp.int32, (nc, n_out), 1)
    ).astype(jnp.bfloat16)
    r2 = lax.dot_general(
        z, f, (((1,), (0,)), ((), ())), preferred_element_type=jnp.float32
    )
    s = (
        lax.broadcasted_iota(jnp.int32, (bm // n_in, bm), 1) // n_in
        == lax.broadcasted_iota(jnp.int32, (bm // n_in, bm), 0)
    ).astype(jnp.bfloat16)
    out = lax.dot_general(
        s, r2.astype(jnp.bfloat16), (((1,), (0,)), ((), ())),
        preferred_element_type=jnp.float32,
    )
    o_ref[...] = out + b_ref[...]


def kernel(cellgene_embedding, genes_oi, weight1, bias1):
    n_cells, n_genes_oi, n_in = cellgene_embedding.shape
    n_out = weight1.shape[2]
    d = n_in * n_out

    info = plsc.get_sparse_core_info()
    num_workers = info.num_cores * info.num_subcores
    rows_per_worker = n_genes_oi // num_workers

    table2d = weight1.reshape(weight1.shape[0], d)
    gather = _make_sc_gather(n_genes_oi, d, num_workers, rows_per_worker,
                             info.num_cores)
    w_rows = gather(table2d, genes_oi.astype(jnp.int32))

    bias2 = bias1.reshape(1, n_out)

    x_perm = jnp.transpose(cellgene_embedding, (0, 2, 1))
    x_mat = x_perm.reshape(n_cells * n_in, n_genes_oi)

    bm = 1024
    grid = (n_cells * n_in // bm,)
    body = functools.partial(_matmul_body, n_in, n_out)
    out = pl.pallas_call(
        body,
        grid=grid,
        in_specs=[
            pl.BlockSpec((bm, n_genes_oi), lambda k: (k, 0)),
            pl.BlockSpec((n_genes_oi, n_in * n_out), lambda k: (0, 0)),
            pl.BlockSpec((1, n_out), lambda k: (0, 0)),
        ],
        out_specs=pl.BlockSpec((bm // n_in, n_out), lambda k: (k, 0)),
        out_shape=jax.ShapeDtypeStruct((n_cells, n_out), jnp.float32),
        compiler_params=pltpu.CompilerParams(
            dimension_semantics=("arbitrary",),
        ),
    )(x_mat, w_rows, bias2)
    return out
